# Initial kernel scaffold; baseline (speedup 1.0000x reference)
import jax
import jax.numpy as jnp
from jax.experimental import pallas as pl


def kernel(x, force_drop, noise):
    raise NotImplementedError("write your pallas kernel here")


# speedup vs baseline: 3.1606x; 3.1606x over previous
import functools

import jax
import jax.numpy as jnp
from jax import lax
from jax.experimental import pallas as pl
from jax.experimental.pallas import tpu as pltpu
from jax.experimental.pallas import tpu_sc as plsc

_BATCH = 64
_SEQ = 1025
_PATCH = _SEQ - 1
_KEEP = _PATCH // 2
_OUT = _KEEP + 1
_DIM = 768
_NW = 32
_BPW = _BATCH // _NW
_CH = 57
_NCH = _OUT // _CH
_L = 16
_NCHUNK = _PATCH // _L


def _select_and_gather(x_hbm, noise_hbm, out_hbm, noise_v, idx2d, rows_v, sem):
    cid = lax.axis_index("c")
    sid = lax.axis_index("s")
    wid = sid * 2 + cid
    lane = lax.iota(jnp.int32, _L)

    for r in range(_BPW):
        b = wid * _BPW + r
        pltpu.sync_copy(noise_hbm.at[b], noise_v)

        def bs_body(_, carry):
            lo, hi = carry
            mid = (lo + hi) >> 1
            pivot = plsc.bitcast(jnp.broadcast_to(mid, (_L,)), jnp.float32)

            def cnt_body(j, acc):
                v = noise_v[pl.ds(j * _L, _L)]
                return acc + (v < pivot).astype(jnp.int32)

            acc = lax.fori_loop(0, _NCHUNK, cnt_body,
                                jnp.zeros((_L,), jnp.int32))
            big = jnp.sum(acc) >= _KEEP
            return jnp.where(big, lo, mid), jnp.where(big, mid, hi)

        lo, _ = lax.fori_loop(0, 30, bs_body,
                              (jnp.int32(0), jnp.int32(1 << 30)))
        thr = plsc.bitcast(jnp.broadcast_to(lo, (_L,)), jnp.float32)

        def lt_body(j, acc):
            v = noise_v[pl.ds(j * _L, _L)]
            return acc + (v < thr).astype(jnp.int32)

        n_lt = jnp.sum(lax.fori_loop(0, _NCHUNK, lt_body,
                                     jnp.zeros((_L,), jnp.int32)))
        need = _KEEP - n_lt

        base_row = b * _SEQ
        idx2d[0, pl.ds(0, _L)] = jnp.broadcast_to(base_row, (_L,))

        def sc_body(j, carry):
            eqc, kc = carry
            v = noise_v[pl.ds(j * _L, _L)]
            lt = v < thr
            eq = v == thr
            eqi = eq.astype(jnp.int32)
            ecum = plsc.cumsum(eqi)
            keep = lt | (eq & ((eqc + ecum - eqi) < need))
            ki = keep.astype(jnp.int32)
            kcum = plsc.cumsum(ki)
            pos = kc + kcum - ki + 1
            val = base_row + j * _L + lane
            plsc.store_scatter(idx2d, [pos // _CH, pos % _CH], val, mask=keep)
            return eqc + ecum[_L - 1], kc + kcum[_L - 1]

        lax.fori_loop(0, _NCHUNK, sc_body, (jnp.int32(0), jnp.int32(0)))

        for c in range(_NCH):
            pltpu.async_copy(x_hbm.at[idx2d.at[c]], rows_v, sem).wait()
            pltpu.sync_copy(rows_v,
                            out_hbm.at[pl.ds(b * _OUT + c * _CH, _CH)])


_patch_dropout_sc = functools.partial(
    pl.kernel,
    out_type=jax.ShapeDtypeStruct((_BATCH * _OUT, _DIM), jnp.float32),
    mesh=plsc.VectorSubcoreMesh(core_axis_name="c", subcore_axis_name="s",
                                num_cores=2, num_subcores=16),
    scratch_types=[
        pltpu.VMEM((_PATCH,), jnp.float32),
        pltpu.VMEM((_NCH, _CH), jnp.int32),
        pltpu.VMEM((_CH, _DIM), jnp.float32),
        pltpu.SemaphoreType.DMA,
    ],
    compiler_params=pltpu.CompilerParams(use_tc_tiling_on_sc=False,
                                         needs_layout_passes=False),
)(_select_and_gather)


def kernel(x, force_drop, noise):
    del force_drop
    b, s, d = x.shape
    out_flat = _patch_dropout_sc(x.reshape(b * s, d), noise)
    return out_flat.reshape(b, _OUT, d)
